# baseline (speedup 1.0000x reference)
import functools

import jax
import jax.numpy as jnp
from jax import lax
from jax.experimental import pallas as pl
from jax.experimental.pallas import tpu as pltpu
from jax.experimental.pallas import tpu_sc as plsc

N = 10000
NP = 10240
D = 128
E = 320000
NC = 2
NS = 16
NW = NC * NS
C = 112
NJ = 96
E_PAD = NW * NJ * C
RPT = NP // NS
DW = 128

_mesh = plsc.VectorSubcoreMesh(core_axis_name="c", subcore_axis_name="s")


@functools.partial(
    pl.kernel,
    out_type=jax.ShapeDtypeStruct((NC, NP, D), jnp.float32),
    mesh=_mesh,
    scratch_types=[
        pltpu.VMEM((NJ // 4, C), jnp.int32),
        pltpu.VMEM((NJ // 4, C), jnp.int32),
        pltpu.VMEM((C, D), jnp.float32),
        pltpu.VMEM((C, D), jnp.float32),
        pltpu.VMEM((C, D), jnp.float32),
        pltpu.VMEM_SHARED((NP, D), jnp.float32),
        pltpu.SemaphoreType.DMA,
        pltpu.SemaphoreType.DMA,
        pltpu.SemaphoreType.DMA,
    ],
)
def _edge_scatter(xs_hbm, src_hbm, dst_hbm, out_hbm,
                  srcb, dstb, rows0, rows1, rows2, acc_sh, sem0, sem1, sem2):
    cid = lax.axis_index("c")
    sid = lax.axis_index("s")
    wid = cid * NS + sid
    roff = pl.multiple_of(sid * RPT, 8)
    JPH = NJ // 4

    zv = jnp.zeros((16,), jnp.float32)

    def zrow(j, carry):
        for k in range(8):
            rows0[j, pl.ds(k * 16, 16)] = zv
        return carry

    lax.fori_loop(0, C, zrow, 0)
    for r in range(RPT // C):
        pltpu.sync_copy(rows0, acc_sh.at[pl.ds(roff + r * C, C)])
    rem = RPT - (RPT // C) * C
    if rem:
        pltpu.sync_copy(rows0.at[pl.ds(0, rem)],
                        acc_sh.at[pl.ds(roff + (RPT // C) * C, rem)])
    plsc.subcore_barrier()

    rows = (rows0, rows1, rows2)
    sems = (sem0, sem1, sem2)

    for ph in range(4):
        coff = pl.multiple_of(wid * NJ + ph * JPH, 8)
        pltpu.sync_copy(src_hbm.at[pl.ds(coff, JPH)], srcb)
        pltpu.sync_copy(dst_hbm.at[pl.ds(coff, JPH)], dstb)

        for b in range(3):
            pltpu.async_copy(xs_hbm.at[srcb.at[b]], rows[b], sems[b])

        def body(t, carry):
            for b in range(3):
                j = 3 * t + b
                pltpu.make_async_copy(
                    xs_hbm.at[srcb.at[j]], rows[b], sems[b]).wait()
                pltpu.sync_copy(rows[b], acc_sh.at[dstb.at[j]], add=True)

                @pl.when(t < JPH // 3 - 1)
                def _():
                    pltpu.async_copy(xs_hbm.at[srcb.at[j + 3]], rows[b], sems[b])

            return carry

        lax.fori_loop(0, JPH // 3, body, 0)

    plsc.subcore_barrier()

    pltpu.sync_copy(acc_sh.at[pl.ds(roff, RPT)],
                    out_hbm.at[cid, pl.ds(roff, RPT)])


@functools.partial(
    pl.kernel,
    out_type=jax.ShapeDtypeStruct((NC, NP, DW), jnp.float32),
    mesh=_mesh,
    scratch_types=[
        pltpu.VMEM((NJ, C), jnp.int32),
        pltpu.VMEM((C, DW), jnp.float32),
        pltpu.VMEM_SHARED((NP, DW), jnp.float32),
    ],
)
def _deg_scatter(dst_hbm, out_hbm, dstb, ones_v, acc_sh):
    cid = lax.axis_index("c")
    sid = lax.axis_index("s")
    wid = cid * NS + sid
    roff = pl.multiple_of(sid * RPT, 8)
    coff = pl.multiple_of(wid * NJ, 8)

    zv = jnp.zeros((16,), jnp.float32)
    ov = jnp.ones((16,), jnp.float32)

    def zrow(j, carry):
        for k in range(8):
            ones_v[j, pl.ds(k * 16, 16)] = zv
        return carry

    lax.fori_loop(0, C, zrow, 0)
    for r in range(RPT // C):
        pltpu.sync_copy(ones_v, acc_sh.at[pl.ds(roff + r * C, C)])
    rem = RPT - (RPT // C) * C
    if rem:
        pltpu.sync_copy(ones_v.at[pl.ds(0, rem)],
                        acc_sh.at[pl.ds(roff + (RPT // C) * C, rem)])

    def orow(j, carry):
        for k in range(8):
            ones_v[j, pl.ds(k * 16, 16)] = ov
        return carry

    lax.fori_loop(0, C, orow, 0)
    pltpu.sync_copy(dst_hbm.at[pl.ds(coff, NJ)], dstb)
    plsc.subcore_barrier()

    def body(j, carry):
        pltpu.sync_copy(ones_v, acc_sh.at[dstb.at[j]], add=True)
        return carry

    lax.fori_loop(0, NJ, body, 0)
    plsc.subcore_barrier()

    pltpu.sync_copy(acc_sh.at[pl.ds(roff, RPT)],
                    out_hbm.at[cid, pl.ds(roff, RPT)])


BT = 2048
_GRID_T = NP // BT
B3 = 2000
_GRID_3 = N // B3


def _tc1_body(deg_ref, x_ref, w_ref, dinv_ref, xs_ref):
    dp = deg_ref[...]
    deg = dp[0, :, 0:1] + dp[1, :, 0:1] + 1.0
    dinv = lax.rsqrt(deg)
    xw = jnp.dot(x_ref[...], w_ref[...], preferred_element_type=jnp.float32)
    dinvb = jnp.broadcast_to(dinv, (BT, D))
    dinv_ref[...] = dinvb
    xs_ref[...] = xw * dinvb


_tc1 = pl.pallas_call(
    _tc1_body,
    grid=(_GRID_T,),
    in_specs=[
        pl.BlockSpec((NC, BT, DW), lambda i: (0, i, 0)),
        pl.BlockSpec((BT, D), lambda i: (i, 0)),
        pl.BlockSpec((D, D), lambda i: (0, 0)),
    ],
    out_specs=[
        pl.BlockSpec((BT, D), lambda i: (i, 0)),
        pl.BlockSpec((BT, D), lambda i: (i, 0)),
    ],
    out_shape=[
        jax.ShapeDtypeStruct((NP, D), jnp.float32),
        jax.ShapeDtypeStruct((NP, D), jnp.float32),
    ],
)


def _tc2_body(p_ref, xs1_ref, dinv_ref, b1_ref, w2_ref, xs2_ref):
    pp = p_ref[...]
    s = pp[0] + pp[1] + xs1_ref[...]
    h = jnp.maximum(dinv_ref[...] * s + b1_ref[...], 0.0)
    hw = jnp.dot(h, w2_ref[...], preferred_element_type=jnp.float32)
    xs2_ref[...] = hw * dinv_ref[...]


_tc2 = pl.pallas_call(
    _tc2_body,
    grid=(_GRID_T,),
    in_specs=[
        pl.BlockSpec((NC, BT, D), lambda i: (0, i, 0)),
        pl.BlockSpec((BT, D), lambda i: (i, 0)),
        pl.BlockSpec((BT, D), lambda i: (i, 0)),
        pl.BlockSpec((1, D), lambda i: (0, 0)),
        pl.BlockSpec((D, D), lambda i: (0, 0)),
    ],
    out_specs=pl.BlockSpec((BT, D), lambda i: (i, 0)),
    out_shape=jax.ShapeDtypeStruct((NP, D), jnp.float32),
)


def _tc3_body(q_ref, xs2_ref, dinv_ref, b2_ref, out_ref):
    qq = q_ref[...]
    s = qq[0] + qq[1] + xs2_ref[...]
    out_ref[...] = dinv_ref[...] * s + b2_ref[...]


_tc3 = pl.pallas_call(
    _tc3_body,
    grid=(_GRID_3,),
    in_specs=[
        pl.BlockSpec((NC, B3, D), lambda i: (0, i, 0)),
        pl.BlockSpec((B3, D), lambda i: (i, 0)),
        pl.BlockSpec((B3, D), lambda i: (i, 0)),
        pl.BlockSpec((1, D), lambda i: (0, 0)),
    ],
    out_specs=pl.BlockSpec((B3, D), lambda i: (i, 0)),
    out_shape=jax.ShapeDtypeStruct((N, D), jnp.float32),
)


def kernel(x, edge_index, W1, b1, W2, b2):
    ei = edge_index.astype(jnp.int32)
    pad = N + (jnp.arange(E_PAD - E, dtype=jnp.int32) % C)
    src = jnp.concatenate([ei[0], pad]).reshape(E_PAD // C, C)
    dst = jnp.concatenate([ei[1], pad]).reshape(E_PAD // C, C)

    degp = _deg_scatter(dst)
    dinv, xs1 = _tc1(degp, x, W1)
    p = _edge_scatter(xs1, src, dst)
    xs2 = _tc2(p, xs1, dinv, b1.reshape(1, D), W2)
    q = _edge_scatter(xs2, src, dst)
    out = _tc3(q, xs2, dinv, b2.reshape(1, D))
    return out


# speedup vs baseline: 1.0131x; 1.0131x over previous
import functools

import jax
import jax.numpy as jnp
from jax import lax
from jax.experimental import pallas as pl
from jax.experimental.pallas import tpu as pltpu
from jax.experimental.pallas import tpu_sc as plsc

N = 10000
NP = 10240
D = 128
E = 320000
NC = 2
NS = 16
NW = NC * NS
C = 128
NJ = 80
E_PAD = NW * NJ * C
RPT = NP // NS
DW = 128

_mesh = plsc.VectorSubcoreMesh(core_axis_name="c", subcore_axis_name="s")


@functools.partial(
    pl.kernel,
    out_type=jax.ShapeDtypeStruct((NC, NP, D), jnp.float32),
    mesh=_mesh,
    scratch_types=[
        pltpu.VMEM((NJ // 2, C), jnp.int32),
        pltpu.VMEM((NJ // 2, C), jnp.int32),
        pltpu.VMEM((C, D), jnp.float32),
        pltpu.VMEM((C, D), jnp.float32),
        pltpu.VMEM_SHARED((NP, D), jnp.float32),
        pltpu.SemaphoreType.DMA,
        pltpu.SemaphoreType.DMA,
    ],
)
def _edge_scatter(xs_hbm, src_hbm, dst_hbm, out_hbm,
                  srcb, dstb, rows0, rows1, acc_sh, sem0, sem1):
    cid = lax.axis_index("c")
    sid = lax.axis_index("s")
    wid = cid * NS + sid
    roff = pl.multiple_of(sid * RPT, 8)
    JPH = NJ // 2

    zv = jnp.zeros((16,), jnp.float32)

    def zrow(j, carry):
        for k in range(8):
            rows0[j, pl.ds(k * 16, 16)] = zv
        return carry

    lax.fori_loop(0, C, zrow, 0)
    for r in range(RPT // C):
        pltpu.sync_copy(rows0, acc_sh.at[pl.ds(roff + r * C, C)])
    plsc.subcore_barrier()

    rows = (rows0, rows1)
    sems = (sem0, sem1)

    for ph in range(2):
        coff = pl.multiple_of(wid * NJ + ph * JPH, 8)
        pltpu.sync_copy(src_hbm.at[pl.ds(coff, JPH)], srcb)
        pltpu.sync_copy(dst_hbm.at[pl.ds(coff, JPH)], dstb)

        pltpu.async_copy(xs_hbm.at[srcb.at[0]], rows0, sem0)
        pltpu.async_copy(xs_hbm.at[srcb.at[1]], rows1, sem1)

        def body(t, carry):
            for b in range(2):
                j = 2 * t + b
                pltpu.make_async_copy(
                    xs_hbm.at[srcb.at[j]], rows[b], sems[b]).wait()
                pltpu.sync_copy(rows[b], acc_sh.at[dstb.at[j]], add=True)

                @pl.when(t < JPH // 2 - 1)
                def _():
                    pltpu.async_copy(xs_hbm.at[srcb.at[j + 2]], rows[b], sems[b])

            return carry

        lax.fori_loop(0, JPH // 2, body, 0)

    plsc.subcore_barrier()

    pltpu.sync_copy(acc_sh.at[pl.ds(roff, RPT)],
                    out_hbm.at[cid, pl.ds(roff, RPT)])


@functools.partial(
    pl.kernel,
    out_type=jax.ShapeDtypeStruct((NC, NP, DW), jnp.float32),
    mesh=_mesh,
    scratch_types=[
        pltpu.VMEM((NJ, C), jnp.int32),
        pltpu.VMEM((C, DW), jnp.float32),
        pltpu.VMEM_SHARED((NP, DW), jnp.float32),
    ],
)
def _deg_scatter(dst_hbm, out_hbm, dstb, ones_v, acc_sh):
    cid = lax.axis_index("c")
    sid = lax.axis_index("s")
    wid = cid * NS + sid
    roff = pl.multiple_of(sid * RPT, 8)
    coff = pl.multiple_of(wid * NJ, 8)

    zv = jnp.zeros((16,), jnp.float32)
    ov = jnp.ones((16,), jnp.float32)

    def zrow(j, carry):
        for k in range(8):
            ones_v[j, pl.ds(k * 16, 16)] = zv
        return carry

    lax.fori_loop(0, C, zrow, 0)
    for r in range(RPT // C):
        pltpu.sync_copy(ones_v, acc_sh.at[pl.ds(roff + r * C, C)])

    def orow(j, carry):
        for k in range(8):
            ones_v[j, pl.ds(k * 16, 16)] = ov
        return carry

    lax.fori_loop(0, C, orow, 0)
    pltpu.sync_copy(dst_hbm.at[pl.ds(coff, NJ)], dstb)
    plsc.subcore_barrier()

    def body(j, carry):
        pltpu.sync_copy(ones_v, acc_sh.at[dstb.at[j]], add=True)
        return carry

    lax.fori_loop(0, NJ, body, 0)
    plsc.subcore_barrier()

    pltpu.sync_copy(acc_sh.at[pl.ds(roff, RPT)],
                    out_hbm.at[cid, pl.ds(roff, RPT)])


BT = 2048
_GRID_T = NP // BT
B3 = 2000
_GRID_3 = N // B3


def _tc1_body(deg_ref, x_ref, w_ref, dinv_ref, xs_ref):
    dp = deg_ref[...]
    deg = dp[0, :, 0:1] + dp[1, :, 0:1] + 1.0
    dinv = lax.rsqrt(deg)
    xw = jnp.dot(x_ref[...], w_ref[...], preferred_element_type=jnp.float32)
    dinvb = jnp.broadcast_to(dinv, (BT, D))
    dinv_ref[...] = dinvb
    xs_ref[...] = xw * dinvb


_tc1 = pl.pallas_call(
    _tc1_body,
    grid=(_GRID_T,),
    in_specs=[
        pl.BlockSpec((NC, BT, DW), lambda i: (0, i, 0)),
        pl.BlockSpec((BT, D), lambda i: (i, 0)),
        pl.BlockSpec((D, D), lambda i: (0, 0)),
    ],
    out_specs=[
        pl.BlockSpec((BT, D), lambda i: (i, 0)),
        pl.BlockSpec((BT, D), lambda i: (i, 0)),
    ],
    out_shape=[
        jax.ShapeDtypeStruct((NP, D), jnp.float32),
        jax.ShapeDtypeStruct((NP, D), jnp.float32),
    ],
)


def _tc2_body(p_ref, xs1_ref, dinv_ref, b1_ref, w2_ref, xs2_ref):
    pp = p_ref[...]
    s = pp[0] + pp[1] + xs1_ref[...]
    h = jnp.maximum(dinv_ref[...] * s + b1_ref[...], 0.0)
    hw = jnp.dot(h, w2_ref[...], preferred_element_type=jnp.float32)
    xs2_ref[...] = hw * dinv_ref[...]


_tc2 = pl.pallas_call(
    _tc2_body,
    grid=(_GRID_T,),
    in_specs=[
        pl.BlockSpec((NC, BT, D), lambda i: (0, i, 0)),
        pl.BlockSpec((BT, D), lambda i: (i, 0)),
        pl.BlockSpec((BT, D), lambda i: (i, 0)),
        pl.BlockSpec((1, D), lambda i: (0, 0)),
        pl.BlockSpec((D, D), lambda i: (0, 0)),
    ],
    out_specs=pl.BlockSpec((BT, D), lambda i: (i, 0)),
    out_shape=jax.ShapeDtypeStruct((NP, D), jnp.float32),
)


def _tc3_body(q_ref, xs2_ref, dinv_ref, b2_ref, out_ref):
    qq = q_ref[...]
    s = qq[0] + qq[1] + xs2_ref[...]
    out_ref[...] = dinv_ref[...] * s + b2_ref[...]


_tc3 = pl.pallas_call(
    _tc3_body,
    grid=(_GRID_3,),
    in_specs=[
        pl.BlockSpec((NC, B3, D), lambda i: (0, i, 0)),
        pl.BlockSpec((B3, D), lambda i: (i, 0)),
        pl.BlockSpec((B3, D), lambda i: (i, 0)),
        pl.BlockSpec((1, D), lambda i: (0, 0)),
    ],
    out_specs=pl.BlockSpec((B3, D), lambda i: (i, 0)),
    out_shape=jax.ShapeDtypeStruct((N, D), jnp.float32),
)


def kernel(x, edge_index, W1, b1, W2, b2):
    ei = edge_index.astype(jnp.int32)
    pad = N + (jnp.arange(E_PAD - E, dtype=jnp.int32) % C)
    src = jnp.concatenate([ei[0], pad]).reshape(E_PAD // C, C)
    dst = jnp.concatenate([ei[1], pad]).reshape(E_PAD // C, C)

    degp = _deg_scatter(dst)
    dinv, xs1 = _tc1(degp, x, W1)
    p = _edge_scatter(xs1, src, dst)
    xs2 = _tc2(p, xs1, dinv, b1.reshape(1, D), W2)
    q = _edge_scatter(xs2, src, dst)
    out = _tc3(q, xs2, dinv, b2.reshape(1, D))
    return out


# speedup vs baseline: 1.0133x; 1.0002x over previous
import functools

import jax
import jax.numpy as jnp
from jax import lax
from jax.experimental import pallas as pl
from jax.experimental.pallas import tpu as pltpu
from jax.experimental.pallas import tpu_sc as plsc

N = 10000
NP = 10240
D = 128
E = 320000
NC = 2
NS = 16
NW = NC * NS
C = 128
NJ = 80
E_PAD = NW * NJ * C
RPT = NP // NS
DW = 128

_mesh = plsc.VectorSubcoreMesh(core_axis_name="c", subcore_axis_name="s",
                               num_cores=NC, num_subcores=NS)


@functools.partial(
    pl.kernel,
    out_type=jax.ShapeDtypeStruct((NC, NP, D), jnp.float32),
    mesh=_mesh,
    scratch_types=[
        pltpu.VMEM((NJ // 2, C), jnp.int32),
        pltpu.VMEM((NJ // 2, C), jnp.int32),
        pltpu.VMEM((C, D), jnp.float32),
        pltpu.VMEM((C, D), jnp.float32),
        pltpu.VMEM_SHARED((NP, D), jnp.float32),
        pltpu.SemaphoreType.DMA,
        pltpu.SemaphoreType.DMA,
    ],
)
def _edge_scatter(xs_hbm, src_hbm, dst_hbm, out_hbm,
                  srcb, dstb, rows0, rows1, acc_sh, sem0, sem1):
    cid = lax.axis_index("c")
    sid = lax.axis_index("s")
    wid = cid * NS + sid
    roff = pl.multiple_of(sid * RPT, 8)
    JPH = NJ // 2

    zv = jnp.zeros((16,), jnp.float32)

    def zrow(j, carry):
        for k in range(8):
            rows0[j, pl.ds(k * 16, 16)] = zv
        return carry

    lax.fori_loop(0, C, zrow, 0)
    for r in range(RPT // C):
        pltpu.sync_copy(rows0, acc_sh.at[pl.ds(roff + r * C, C)])
    plsc.subcore_barrier()

    rows = (rows0, rows1)
    sems = (sem0, sem1)

    for ph in range(2):
        coff = pl.multiple_of(wid * NJ + ph * JPH, 8)
        pltpu.sync_copy(src_hbm.at[pl.ds(coff, JPH)], srcb)
        pltpu.sync_copy(dst_hbm.at[pl.ds(coff, JPH)], dstb)

        pltpu.async_copy(xs_hbm.at[srcb.at[0]], rows0, sem0)
        pltpu.async_copy(xs_hbm.at[srcb.at[1]], rows1, sem1)

        def body(t, carry):
            for b in range(2):
                j = 2 * t + b
                pltpu.make_async_copy(
                    xs_hbm.at[srcb.at[j]], rows[b], sems[b]).wait()
                pltpu.sync_copy(rows[b], acc_sh.at[dstb.at[j]], add=True)

                @pl.when(t < JPH // 2 - 1)
                def _():
                    pltpu.async_copy(xs_hbm.at[srcb.at[j + 2]], rows[b], sems[b])

            return carry

        lax.fori_loop(0, JPH // 2, body, 0)

    plsc.subcore_barrier()

    pltpu.sync_copy(acc_sh.at[pl.ds(roff, RPT)],
                    out_hbm.at[cid, pl.ds(roff, RPT)])


@functools.partial(
    pl.kernel,
    out_type=jax.ShapeDtypeStruct((NC, NP, DW), jnp.float32),
    mesh=_mesh,
    scratch_types=[
        pltpu.VMEM((NJ, C), jnp.int32),
        pltpu.VMEM((C, DW), jnp.float32),
        pltpu.VMEM_SHARED((NP, DW), jnp.float32),
    ],
)
def _deg_scatter(dst_hbm, out_hbm, dstb, ones_v, acc_sh):
    cid = lax.axis_index("c")
    sid = lax.axis_index("s")
    wid = cid * NS + sid
    roff = pl.multiple_of(sid * RPT, 8)
    coff = pl.multiple_of(wid * NJ, 8)

    zv = jnp.zeros((16,), jnp.float32)
    ov = jnp.ones((16,), jnp.float32)

    def zrow(j, carry):
        for k in range(8):
            ones_v[j, pl.ds(k * 16, 16)] = zv
        return carry

    lax.fori_loop(0, C, zrow, 0)
    for r in range(RPT // C):
        pltpu.sync_copy(ones_v, acc_sh.at[pl.ds(roff + r * C, C)])

    def orow(j, carry):
        for k in range(8):
            ones_v[j, pl.ds(k * 16, 16)] = ov
        return carry

    lax.fori_loop(0, C, orow, 0)
    pltpu.sync_copy(dst_hbm.at[pl.ds(coff, NJ)], dstb)
    plsc.subcore_barrier()

    def body(j, carry):
        pltpu.sync_copy(ones_v, acc_sh.at[dstb.at[j]], add=True)
        return carry

    lax.fori_loop(0, NJ, body, 0)
    plsc.subcore_barrier()

    pltpu.sync_copy(acc_sh.at[pl.ds(roff, RPT)],
                    out_hbm.at[cid, pl.ds(roff, RPT)])


BT = 2048
_GRID_T = NP // BT
B3 = 2000
_GRID_3 = N // B3


def _tc1_body(deg_ref, x_ref, w_ref, dinv_ref, xs_ref):
    dp = deg_ref[...]
    deg = dp[0, :, 0:1] + dp[1, :, 0:1] + 1.0
    dinv = lax.rsqrt(deg)
    xw = jnp.dot(x_ref[...], w_ref[...], preferred_element_type=jnp.float32)
    dinvb = jnp.broadcast_to(dinv, (BT, D))
    dinv_ref[...] = dinvb
    xs_ref[...] = xw * dinvb


_tc1 = pl.pallas_call(
    _tc1_body,
    grid=(_GRID_T,),
    in_specs=[
        pl.BlockSpec((NC, BT, DW), lambda i: (0, i, 0)),
        pl.BlockSpec((BT, D), lambda i: (i, 0)),
        pl.BlockSpec((D, D), lambda i: (0, 0)),
    ],
    out_specs=[
        pl.BlockSpec((BT, D), lambda i: (i, 0)),
        pl.BlockSpec((BT, D), lambda i: (i, 0)),
    ],
    out_shape=[
        jax.ShapeDtypeStruct((NP, D), jnp.float32),
        jax.ShapeDtypeStruct((NP, D), jnp.float32),
    ],
)


def _tc2_body(p_ref, xs1_ref, dinv_ref, b1_ref, w2_ref, xs2_ref):
    pp = p_ref[...]
    s = pp[0] + pp[1] + xs1_ref[...]
    h = jnp.maximum(dinv_ref[...] * s + b1_ref[...], 0.0)
    hw = jnp.dot(h, w2_ref[...], preferred_element_type=jnp.float32)
    xs2_ref[...] = hw * dinv_ref[...]


_tc2 = pl.pallas_call(
    _tc2_body,
    grid=(_GRID_T,),
    in_specs=[
        pl.BlockSpec((NC, BT, D), lambda i: (0, i, 0)),
        pl.BlockSpec((BT, D), lambda i: (i, 0)),
        pl.BlockSpec((BT, D), lambda i: (i, 0)),
        pl.BlockSpec((1, D), lambda i: (0, 0)),
        pl.BlockSpec((D, D), lambda i: (0, 0)),
    ],
    out_specs=pl.BlockSpec((BT, D), lambda i: (i, 0)),
    out_shape=jax.ShapeDtypeStruct((NP, D), jnp.float32),
)


def _tc3_body(q_ref, xs2_ref, dinv_ref, b2_ref, out_ref):
    qq = q_ref[...]
    s = qq[0] + qq[1] + xs2_ref[...]
    out_ref[...] = dinv_ref[...] * s + b2_ref[...]


_tc3 = pl.pallas_call(
    _tc3_body,
    grid=(_GRID_3,),
    in_specs=[
        pl.BlockSpec((NC, B3, D), lambda i: (0, i, 0)),
        pl.BlockSpec((B3, D), lambda i: (i, 0)),
        pl.BlockSpec((B3, D), lambda i: (i, 0)),
        pl.BlockSpec((1, D), lambda i: (0, 0)),
    ],
    out_specs=pl.BlockSpec((B3, D), lambda i: (i, 0)),
    out_shape=jax.ShapeDtypeStruct((N, D), jnp.float32),
)


def kernel(x, edge_index, W1, b1, W2, b2):
    ei = edge_index.astype(jnp.int32)
    pad = N + (jnp.arange(E_PAD - E, dtype=jnp.int32) % C)
    src = jnp.concatenate([ei[0], pad]).reshape(E_PAD // C, C)
    dst = jnp.concatenate([ei[1], pad]).reshape(E_PAD // C, C)

    degp = _deg_scatter(dst)
    dinv, xs1 = _tc1(degp, x, W1)
    p = _edge_scatter(xs1, src, dst)
    xs2 = _tc2(p, xs1, dinv, b1.reshape(1, D), W2)
    q = _edge_scatter(xs2, src, dst)
    out = _tc3(q, xs2, dinv, b2.reshape(1, D))
    return out
